# Initial kernel scaffold; baseline (speedup 1.0000x reference)
import jax
import jax.numpy as jnp
from jax.experimental import pallas as pl


def kernel(x_adresse, x_batiment, x_parcelle, ea_ab, ea_ba, ea_bp, ea_pb, Wl, Wr, We, att, bias, Wlin, blin, ei_ab, ei_ba, ei_bp, ei_pb):
    raise NotImplementedError("write your pallas kernel here")


# speedup vs baseline: 6.0520x; 6.0520x over previous
import functools

import jax
import jax.numpy as jnp
from jax import lax
from jax.experimental import pallas as pl
from jax.experimental.pallas import tpu as pltpu
from jax.experimental.pallas import tpu_sc as plsc

N = 10000
C = 128
ED = 16
OUT = 64

NC = 2
NS = 16
NW = NC * NS
LANES = 16
CB = C // LANES
CW = C + LANES
B = 64
NP = 10240
ROWS_PER_TILE = NP // NS


def _sc_edge_body(hs_hbm, hd_hbm, he_hbm, src_hbm, dst_hbm, att_hbm,
                  out_hbm, att_v, src_v, dst_v, hs_g, hd_g, he_g, wext,
                  acc_sp, sem, nblk):
  c = lax.axis_index("c")
  s = lax.axis_index("s")
  wg = s * NC + c

  zero16 = jnp.zeros((LANES,), jnp.float32)
  def _zero_row(i, _):
    for cb in range(CW // LANES):
      wext[i, pl.ds(cb * LANES, LANES)] = zero16
    return 0
  lax.fori_loop(0, B, _zero_row, 0)
  row0 = s * ROWS_PER_TILE
  nfull = ROWS_PER_TILE // B
  for k in range(nfull):
    pltpu.sync_copy(wext, acc_sp.at[pl.ds(row0 + k * B, B)])
  plsc.subcore_barrier()

  pltpu.sync_copy(att_hbm, att_v)
  att_c = [att_v[cb] for cb in range(CB)]
  lane = lax.iota(jnp.int32, LANES)
  oh0 = jnp.where(lane == 0, 1.0, 0.0).astype(jnp.float32)

  nblk_w = (nblk // NW) + jnp.where(wg < (nblk % NW), 1, 0)

  def _block(j, _):
    base = (wg + j * NW) * B
    pltpu.sync_copy(src_hbm.at[pl.ds(base, B)], src_v)
    pltpu.sync_copy(dst_hbm.at[pl.ds(base, B)], dst_v)
    d1 = pltpu.async_copy(hs_hbm.at[src_v], hs_g, sem)
    d2 = pltpu.async_copy(hd_hbm.at[dst_v], hd_g, sem)
    pltpu.sync_copy(he_hbm.at[pl.ds(base, B)], he_g)
    d1.wait()
    d2.wait()

    def _edge(i, _):
      hs_c = []
      acc = jnp.zeros((LANES,), jnp.float32)
      for cb in range(CB):
        h = hs_g[i, pl.ds(cb * LANES, LANES)]
        hs_c.append(h)
        v = h + hd_g[i, pl.ds(cb * LANES, LANES)] + he_g[i, pl.ds(cb * LANES, LANES)]
        v = jnp.maximum(v, 0.2 * v)
        acc = acc + v * att_c[cb]
      e = jnp.sum(acc)
      ex = jnp.exp(jnp.full((LANES,), e, jnp.float32))
      for cb in range(CB):
        wext[i, pl.ds(cb * LANES, LANES)] = hs_c[cb] * ex
      wext[i, pl.ds(C, LANES)] = ex * oh0
      return 0
    lax.fori_loop(0, B, _edge, 0)

    pltpu.sync_copy(wext, acc_sp.at[dst_v], add=True)
    return 0
  lax.fori_loop(0, nblk_w, _block, 0)

  plsc.subcore_barrier()
  pltpu.sync_copy(acc_sp.at[pl.ds(row0, ROWS_PER_TILE)],
                  out_hbm.at[c, pl.ds(row0, ROWS_PER_TILE)])


def _sc_edge_pass(hs, hd, he, src, dst, att):
  E = src.shape[0]
  nblk = E // B
  mesh = plsc.VectorSubcoreMesh(core_axis_name="c", subcore_axis_name="s")
  body = functools.partial(_sc_edge_body, nblk=nblk)
  f = pl.kernel(
      body,
      out_type=jax.ShapeDtypeStruct((NC, NP, CW), jnp.float32),
      mesh=mesh,
      compiler_params=pltpu.CompilerParams(
          needs_layout_passes=False, use_tc_tiling_on_sc=False),
      scratch_types=[
          pltpu.VMEM((CB, LANES), jnp.float32),
          pltpu.VMEM((B,), jnp.int32),
          pltpu.VMEM((B,), jnp.int32),
          pltpu.VMEM((B, C), jnp.float32),
          pltpu.VMEM((B, C), jnp.float32),
          pltpu.VMEM((B, C), jnp.float32),
          pltpu.VMEM((B, CW), jnp.float32),
          pltpu.VMEM_SHARED((NP, CW), jnp.float32),
          pltpu.SemaphoreType.DMA,
      ],
  )
  return f(hs, hd, he, src, dst, att.reshape(CB, LANES))


def _mm_body(x_ref, w_ref, b_ref, o_ref):
  o_ref[...] = jnp.dot(x_ref[...], w_ref[...],
                       preferred_element_type=jnp.float32) + b_ref[...]


def _tc_matmul(x, w, b=None, bm=400):
  M, K = x.shape
  Kn = w.shape[1]
  if b is None:
    b = jnp.zeros((1, Kn), jnp.float32)
  else:
    b = b.reshape(1, Kn)
  grid = M // bm
  return pl.pallas_call(
      _mm_body,
      grid=(grid,),
      in_specs=[
          pl.BlockSpec((bm, K), lambda i: (i, 0)),
          pl.BlockSpec((K, Kn), lambda i: (0, 0)),
          pl.BlockSpec((1, Kn), lambda i: (0, 0)),
      ],
      out_specs=pl.BlockSpec((bm, Kn), lambda i: (i, 0)),
      out_shape=jax.ShapeDtypeStruct((M, Kn), jnp.float32),
  )(x, w, b)


def _fin1_body(a_ref, b_ref, o_ref):
  num = a_ref[0, :, :C] + a_ref[1, :, :C]
  den = a_ref[0, :, C:C + 1] + a_ref[1, :, C:C + 1]
  o_ref[...] = jnp.maximum(num / (den + 1e-16) + b_ref[...], 0.0)


def _finalize1(acc, bias, bm=400):
  return pl.pallas_call(
      _fin1_body,
      grid=(N // bm,),
      in_specs=[
          pl.BlockSpec((NC, bm, CW), lambda i: (0, i, 0)),
          pl.BlockSpec((1, C), lambda i: (0, 0)),
      ],
      out_specs=pl.BlockSpec((bm, C), lambda i: (i, 0)),
      out_shape=jax.ShapeDtypeStruct((N, C), jnp.float32),
  )(acc, bias.reshape(1, C))


def _fin2_body(a_ref, c_ref, b1_ref, b2_ref, o_ref):
  num1 = a_ref[0, :, :C] + a_ref[1, :, :C]
  den1 = a_ref[0, :, C:C + 1] + a_ref[1, :, C:C + 1]
  num2 = c_ref[0, :, :C] + c_ref[1, :, :C]
  den2 = c_ref[0, :, C:C + 1] + c_ref[1, :, C:C + 1]
  o_ref[...] = jnp.maximum(
      num1 / (den1 + 1e-16) + b1_ref[...] +
      num2 / (den2 + 1e-16) + b2_ref[...], 0.0)


def _finalize2(acc1, acc2, bias1, bias2, bm=400):
  return pl.pallas_call(
      _fin2_body,
      grid=(N // bm,),
      in_specs=[
          pl.BlockSpec((NC, bm, CW), lambda i: (0, i, 0)),
          pl.BlockSpec((NC, bm, CW), lambda i: (0, i, 0)),
          pl.BlockSpec((1, C), lambda i: (0, 0)),
          pl.BlockSpec((1, C), lambda i: (0, 0)),
      ],
      out_specs=pl.BlockSpec((bm, C), lambda i: (i, 0)),
      out_shape=jax.ShapeDtypeStruct((N, C), jnp.float32),
  )(acc1, acc2, bias1.reshape(1, C), bias2.reshape(1, C))


def kernel(x_adresse, x_batiment, x_parcelle, ea_ab, ea_ba, ea_bp, ea_pb,
           Wl, Wr, We, att, bias, Wlin, blin, ei_ab, ei_ba, ei_bp, ei_pb):
  xa, xb, xp = x_adresse, x_batiment, x_parcelle
  src_ab, dst_ab = ei_ab[0].astype(jnp.int32), ei_ab[1].astype(jnp.int32)
  src_ba, dst_ba = ei_ba[0].astype(jnp.int32), ei_ba[1].astype(jnp.int32)
  src_bp, dst_bp = ei_bp[0].astype(jnp.int32), ei_bp[1].astype(jnp.int32)
  src_pb, dst_pb = ei_pb[0].astype(jnp.int32), ei_pb[1].astype(jnp.int32)

  for l in range(2):
    wa = jnp.concatenate([Wl[l, 0], Wr[l, 1]], axis=1)
    wb = jnp.concatenate([Wr[l, 0], Wl[l, 1], Wl[l, 2], Wr[l, 3]], axis=1)
    wp = jnp.concatenate([Wr[l, 2], Wl[l, 3]], axis=1)
    ha = _tc_matmul(xa, wa)
    hb = _tc_matmul(xb, wb)
    hp = _tc_matmul(xp, wp)
    hs0, hd1 = ha[:, :C], ha[:, C:]
    hd0, hs1, hs2, hd3 = hb[:, :C], hb[:, C:2 * C], hb[:, 2 * C:3 * C], hb[:, 3 * C:]
    hd2, hs3 = hp[:, :C], hp[:, C:]

    he0 = _tc_matmul(ea_ab, We[l, 0], bm=1000)
    he1 = _tc_matmul(ea_ba, We[l, 1], bm=1000)
    he2 = _tc_matmul(ea_bp, We[l, 2], bm=1000)
    he3 = _tc_matmul(ea_pb, We[l, 3], bm=1000)

    acc0 = _sc_edge_pass(hs0, hd0, he0, src_ab, dst_ab, att[l, 0])
    acc1 = _sc_edge_pass(hs1, hd1, he1, src_ba, dst_ba, att[l, 1])
    acc2 = _sc_edge_pass(hs2, hd2, he2, src_bp, dst_bp, att[l, 2])
    acc3 = _sc_edge_pass(hs3, hd3, he3, src_pb, dst_pb, att[l, 3])

    xa = _finalize1(acc1, bias[l, 1])
    xp = _finalize1(acc2, bias[l, 2])
    xb = _finalize2(acc0, acc3, bias[l, 0], bias[l, 3])

  ya = _tc_matmul(xa, Wlin[0], blin[0])
  yb = _tc_matmul(xb, Wlin[1], blin[1])
  yp = _tc_matmul(xp, Wlin[2], blin[2])
  return (ya, yb, yp)


# speedup vs baseline: 7.5228x; 1.2430x over previous
import functools

import jax
import jax.numpy as jnp
from jax import lax
from jax.experimental import pallas as pl
from jax.experimental.pallas import tpu as pltpu
from jax.experimental.pallas import tpu_sc as plsc

N = 10000
C = 128
ED = 16
OUT = 64

NC = 2
NS = 16
NW = NC * NS
LANES = 16
CB = C // LANES
CW = C + LANES
B = 64
NP = 10240
ROWS_PER_TILE = NP // NS


def _sc_edge_body(hs_hbm, hd_hbm, he_hbm, src_hbm, dst_hbm, att_hbm,
                  out_hbm, att_v, src_v, dst_v, hs_g, hd_g, he_g, wext,
                  acc_sp, sem, nblk):
  c = lax.axis_index("c")
  s = lax.axis_index("s")
  wg = s * NC + c

  zero16 = jnp.zeros((LANES,), jnp.float32)
  def _zero_row(i, _):
    for cb in range(CW // LANES):
      wext[i, pl.ds(cb * LANES, LANES)] = zero16
    return 0
  lax.fori_loop(0, B, _zero_row, 0)
  row0 = s * ROWS_PER_TILE
  nfull = ROWS_PER_TILE // B
  for k in range(nfull):
    pltpu.sync_copy(wext, acc_sp.at[pl.ds(row0 + k * B, B)])
  plsc.subcore_barrier()

  pltpu.sync_copy(att_hbm, att_v)
  att_c = [att_v[cb] for cb in range(CB)]
  lane = lax.iota(jnp.int32, LANES)
  oh0 = jnp.where(lane == 0, 1.0, 0.0).astype(jnp.float32)

  nblk_w = (nblk // NW) + jnp.where(wg < (nblk % NW), 1, 0)

  def _block(j, _):
    base = (wg + j * NW) * B
    pltpu.sync_copy(src_hbm.at[pl.ds(base, B)], src_v)
    pltpu.sync_copy(dst_hbm.at[pl.ds(base, B)], dst_v)
    d1 = pltpu.async_copy(hs_hbm.at[src_v], hs_g, sem)
    d2 = pltpu.async_copy(hd_hbm.at[dst_v], hd_g, sem)
    pltpu.sync_copy(he_hbm.at[pl.ds(base, B)], he_g)
    d1.wait()
    d2.wait()

    @plsc.parallel_loop(0, B, unroll=8)
    def _edge(i):
      hs_c = []
      acc = jnp.zeros((LANES,), jnp.float32)
      for cb in range(CB):
        h = hs_g[i, pl.ds(cb * LANES, LANES)]
        hs_c.append(h)
        v = h + hd_g[i, pl.ds(cb * LANES, LANES)] + he_g[i, pl.ds(cb * LANES, LANES)]
        v = jnp.maximum(v, 0.2 * v)
        acc = acc + v * att_c[cb]
      e = jnp.sum(acc)
      ex = jnp.exp(jnp.full((LANES,), e, jnp.float32))
      for cb in range(CB):
        wext[i, pl.ds(cb * LANES, LANES)] = hs_c[cb] * ex
      wext[i, pl.ds(C, LANES)] = ex * oh0

    pltpu.sync_copy(wext, acc_sp.at[dst_v], add=True)
    return 0
  lax.fori_loop(0, nblk_w, _block, 0)

  plsc.subcore_barrier()
  pltpu.sync_copy(acc_sp.at[pl.ds(row0, ROWS_PER_TILE)],
                  out_hbm.at[c, pl.ds(row0, ROWS_PER_TILE)])


def _sc_edge_pass(hs, hd, he, src, dst, att):
  E = src.shape[0]
  nblk = E // B
  mesh = plsc.VectorSubcoreMesh(core_axis_name="c", subcore_axis_name="s")
  body = functools.partial(_sc_edge_body, nblk=nblk)
  f = pl.kernel(
      body,
      out_type=jax.ShapeDtypeStruct((NC, NP, CW), jnp.float32),
      mesh=mesh,
      compiler_params=pltpu.CompilerParams(
          needs_layout_passes=False, use_tc_tiling_on_sc=False),
      scratch_types=[
          pltpu.VMEM((CB, LANES), jnp.float32),
          pltpu.VMEM((B,), jnp.int32),
          pltpu.VMEM((B,), jnp.int32),
          pltpu.VMEM((B, C), jnp.float32),
          pltpu.VMEM((B, C), jnp.float32),
          pltpu.VMEM((B, C), jnp.float32),
          pltpu.VMEM((B, CW), jnp.float32),
          pltpu.VMEM_SHARED((NP, CW), jnp.float32),
          pltpu.SemaphoreType.DMA,
      ],
  )
  return f(hs, hd, he, src, dst, att.reshape(CB, LANES))


def _mm_body(x_ref, w_ref, b_ref, o_ref):
  o_ref[...] = jnp.dot(x_ref[...], w_ref[...],
                       preferred_element_type=jnp.float32) + b_ref[...]


def _tc_matmul(x, w, b=None, bm=400):
  M, K = x.shape
  Kn = w.shape[1]
  if b is None:
    b = jnp.zeros((1, Kn), jnp.float32)
  else:
    b = b.reshape(1, Kn)
  grid = M // bm
  return pl.pallas_call(
      _mm_body,
      grid=(grid,),
      in_specs=[
          pl.BlockSpec((bm, K), lambda i: (i, 0)),
          pl.BlockSpec((K, Kn), lambda i: (0, 0)),
          pl.BlockSpec((1, Kn), lambda i: (0, 0)),
      ],
      out_specs=pl.BlockSpec((bm, Kn), lambda i: (i, 0)),
      out_shape=jax.ShapeDtypeStruct((M, Kn), jnp.float32),
  )(x, w, b)


def _fin1_body(a_ref, b_ref, o_ref):
  num = a_ref[0, :, :C] + a_ref[1, :, :C]
  den = a_ref[0, :, C:C + 1] + a_ref[1, :, C:C + 1]
  o_ref[...] = jnp.maximum(num / (den + 1e-16) + b_ref[...], 0.0)


def _finalize1(acc, bias, bm=400):
  return pl.pallas_call(
      _fin1_body,
      grid=(N // bm,),
      in_specs=[
          pl.BlockSpec((NC, bm, CW), lambda i: (0, i, 0)),
          pl.BlockSpec((1, C), lambda i: (0, 0)),
      ],
      out_specs=pl.BlockSpec((bm, C), lambda i: (i, 0)),
      out_shape=jax.ShapeDtypeStruct((N, C), jnp.float32),
  )(acc, bias.reshape(1, C))


def _fin2_body(a_ref, c_ref, b1_ref, b2_ref, o_ref):
  num1 = a_ref[0, :, :C] + a_ref[1, :, :C]
  den1 = a_ref[0, :, C:C + 1] + a_ref[1, :, C:C + 1]
  num2 = c_ref[0, :, :C] + c_ref[1, :, :C]
  den2 = c_ref[0, :, C:C + 1] + c_ref[1, :, C:C + 1]
  o_ref[...] = jnp.maximum(
      num1 / (den1 + 1e-16) + b1_ref[...] +
      num2 / (den2 + 1e-16) + b2_ref[...], 0.0)


def _finalize2(acc1, acc2, bias1, bias2, bm=400):
  return pl.pallas_call(
      _fin2_body,
      grid=(N // bm,),
      in_specs=[
          pl.BlockSpec((NC, bm, CW), lambda i: (0, i, 0)),
          pl.BlockSpec((NC, bm, CW), lambda i: (0, i, 0)),
          pl.BlockSpec((1, C), lambda i: (0, 0)),
          pl.BlockSpec((1, C), lambda i: (0, 0)),
      ],
      out_specs=pl.BlockSpec((bm, C), lambda i: (i, 0)),
      out_shape=jax.ShapeDtypeStruct((N, C), jnp.float32),
  )(acc1, acc2, bias1.reshape(1, C), bias2.reshape(1, C))


def kernel(x_adresse, x_batiment, x_parcelle, ea_ab, ea_ba, ea_bp, ea_pb,
           Wl, Wr, We, att, bias, Wlin, blin, ei_ab, ei_ba, ei_bp, ei_pb):
  xa, xb, xp = x_adresse, x_batiment, x_parcelle
  src_ab, dst_ab = ei_ab[0].astype(jnp.int32), ei_ab[1].astype(jnp.int32)
  src_ba, dst_ba = ei_ba[0].astype(jnp.int32), ei_ba[1].astype(jnp.int32)
  src_bp, dst_bp = ei_bp[0].astype(jnp.int32), ei_bp[1].astype(jnp.int32)
  src_pb, dst_pb = ei_pb[0].astype(jnp.int32), ei_pb[1].astype(jnp.int32)

  for l in range(2):
    wa = jnp.concatenate([Wl[l, 0], Wr[l, 1]], axis=1)
    wb = jnp.concatenate([Wr[l, 0], Wl[l, 1], Wl[l, 2], Wr[l, 3]], axis=1)
    wp = jnp.concatenate([Wr[l, 2], Wl[l, 3]], axis=1)
    ha = _tc_matmul(xa, wa)
    hb = _tc_matmul(xb, wb)
    hp = _tc_matmul(xp, wp)
    hs0, hd1 = ha[:, :C], ha[:, C:]
    hd0, hs1, hs2, hd3 = hb[:, :C], hb[:, C:2 * C], hb[:, 2 * C:3 * C], hb[:, 3 * C:]
    hd2, hs3 = hp[:, :C], hp[:, C:]

    he0 = _tc_matmul(ea_ab, We[l, 0], bm=1000)
    he1 = _tc_matmul(ea_ba, We[l, 1], bm=1000)
    he2 = _tc_matmul(ea_bp, We[l, 2], bm=1000)
    he3 = _tc_matmul(ea_pb, We[l, 3], bm=1000)

    acc0 = _sc_edge_pass(hs0, hd0, he0, src_ab, dst_ab, att[l, 0])
    acc1 = _sc_edge_pass(hs1, hd1, he1, src_ba, dst_ba, att[l, 1])
    acc2 = _sc_edge_pass(hs2, hd2, he2, src_bp, dst_bp, att[l, 2])
    acc3 = _sc_edge_pass(hs3, hd3, he3, src_pb, dst_pb, att[l, 3])

    xa = _finalize1(acc1, bias[l, 1])
    xp = _finalize1(acc2, bias[l, 2])
    xb = _finalize2(acc0, acc3, bias[l, 0], bias[l, 3])

  ya = _tc_matmul(xa, Wlin[0], blin[0])
  yb = _tc_matmul(xb, Wlin[1], blin[1])
  yp = _tc_matmul(xp, Wlin[2], blin[2])
  return (ya, yb, yp)


# speedup vs baseline: 12.1095x; 1.6097x over previous
import functools

import jax
import jax.numpy as jnp
from jax import lax
from jax.experimental import pallas as pl
from jax.experimental.pallas import tpu as pltpu
from jax.experimental.pallas import tpu_sc as plsc

N = 10000
C = 128
ED = 16
OUT = 64

NC = 2
NS = 16
NW = NC * NS
LANES = 16
CB = C // LANES
CW = C + LANES
B = 32
NP = 10240
ROWS_PER_TILE = NP // NS


def _sc_edge_body(hs_hbm, hd_hbm, he_hbm, src_hbm, dst_hbm, att_hbm,
                  out_hbm, att_v, src_v0, src_v1, dst_v0, dst_v1, dst_v2,
                  dst_v3, hs_g0, hs_g1, hd_g0, hd_g1, he_g0, he_g1,
                  wext0, wext1, acc_sp, gsem0, gsem1, isem0, isem1, ssem0,
                  nblk):
  c = lax.axis_index("c")
  s = lax.axis_index("s")
  wg = s * NC + c
  src_v = [src_v0, src_v1]
  dst_v = [dst_v0, dst_v1, dst_v2, dst_v3]
  hs_g = [hs_g0, hs_g1]
  hd_g = [hd_g0, hd_g1]
  he_g = [he_g0, he_g1]
  wext = [wext0, wext1]
  gsem = [gsem0, gsem1]
  isem = [isem0, isem1]

  zero16 = jnp.zeros((LANES,), jnp.float32)
  def _zero_row(i, _):
    for cb in range(CW // LANES):
      wext0[i, pl.ds(cb * LANES, LANES)] = zero16
    return 0
  lax.fori_loop(0, B, _zero_row, 0)
  row0 = s * ROWS_PER_TILE
  for k in range(ROWS_PER_TILE // B):
    pltpu.sync_copy(wext0, acc_sp.at[pl.ds(row0 + k * B, B)])
  plsc.subcore_barrier()

  pltpu.sync_copy(att_hbm, att_v)
  att_c = [att_v[cb] for cb in range(CB)]
  lane = lax.iota(jnp.int32, LANES)
  oh0 = jnp.where(lane == 0, 1.0, 0.0).astype(jnp.float32)

  nblk_w = -(-nblk // NW)
  quads = nblk_w // 4
  rem = nblk_w % 4

  def ebase(j):
    return jnp.minimum(wg + j * NW, nblk - 1) * B

  def start_idx(j, p, d):
    pltpu.async_copy(src_hbm.at[pl.ds(ebase(j), B)], src_v[p], isem[p])
    pltpu.async_copy(dst_hbm.at[pl.ds(ebase(j), B)], dst_v[d], isem[p])

  def wait_idx(p, d):
    pltpu.make_async_copy(src_hbm.at[pl.ds(0, B)], src_v[p], isem[p]).wait()
    pltpu.make_async_copy(dst_hbm.at[pl.ds(0, B)], dst_v[d], isem[p]).wait()

  def start_gathers(j, p, d):
    pltpu.async_copy(hs_hbm.at[src_v[p]], hs_g[p], gsem[p])
    pltpu.async_copy(hd_hbm.at[dst_v[d]], hd_g[p], gsem[p])
    pltpu.async_copy(he_hbm.at[pl.ds(ebase(j), B)], he_g[p], gsem[p])

  def wait_gathers(p):
    pltpu.make_async_copy(hs_hbm.at[src_v[p]], hs_g[p], gsem[p]).wait()
    pltpu.make_async_copy(hd_hbm.at[pl.ds(0, B)], hd_g[p], gsem[p]).wait()
    pltpu.make_async_copy(he_hbm.at[pl.ds(0, B)], he_g[p], gsem[p]).wait()

  def compute(p, jv):
    hs_b, hd_b, he_b, wext_b = hs_g[p], hd_g[p], he_g[p], wext[p]
    ok = (wg + jv * NW) < nblk

    @plsc.parallel_loop(0, B, unroll=8)
    def _edge(i):
      hs_c = []
      acc = jnp.zeros((LANES,), jnp.float32)
      for cb in range(CB):
        h = hs_b[i, pl.ds(cb * LANES, LANES)]
        hs_c.append(h)
        v = h + hd_b[i, pl.ds(cb * LANES, LANES)] + he_b[i, pl.ds(cb * LANES, LANES)]
        v = jnp.maximum(v, 0.2 * v)
        acc = acc + v * att_c[cb]
      e = jnp.sum(acc)
      e = jnp.where(ok, e, jnp.float32(-1e30))
      ex = jnp.exp(jnp.full((LANES,), e, jnp.float32))
      for cb in range(CB):
        wext_b[i, pl.ds(cb * LANES, LANES)] = hs_c[cb] * ex
      wext_b[i, pl.ds(C, LANES)] = ex * oh0

  def pair(j0, s0, s1, n0, n1, pf2, pf3):
    wait_idx(1, s1)
    start_gathers(j0 + 1, 1, s1)
    wait_gathers(0)
    if pf2:
      start_idx(j0 + 2, 0, n0)
    compute(0, j0)
    d0 = pltpu.async_copy(wext[0], acc_sp.at[dst_v[s0]], ssem0, add=True)
    wait_gathers(1)
    if pf3:
      start_idx(j0 + 3, 1, n1)
    if pf2:
      wait_idx(0, n0)
      start_gathers(j0 + 2, 0, n0)
    compute(1, j0 + 1)
    d0.wait()
    pltpu.sync_copy(wext[1], acc_sp.at[dst_v[s1]], add=True)

  pltpu.sync_copy(src_hbm.at[pl.ds(ebase(0), B)], src_v[0])
  pltpu.sync_copy(dst_hbm.at[pl.ds(ebase(0), B)], dst_v[0])
  start_gathers(0, 0, 0)
  start_idx(1, 1, 1)

  def _quad(u, _):
    pair(4 * u, 0, 1, 2, 3, True, True)
    pair(4 * u + 2, 2, 3, 0, 1, True, True)
    return 0
  lax.fori_loop(0, quads - 1, _quad, 0)
  jq = 4 * (quads - 1)
  pair(jq, 0, 1, 2, 3, True, True)
  pair(jq + 2, 2, 3, 0, 1, rem >= 1, rem >= 2)

  base = 4 * quads
  if rem >= 1:
    wait_gathers(0)
    compute(0, base)
    pltpu.sync_copy(wext[0], acc_sp.at[dst_v[0]], add=True)
  if rem >= 2:
    wait_idx(1, 1)
    start_gathers(base + 1, 1, 1)
    wait_gathers(1)
    compute(1, base + 1)
    pltpu.sync_copy(wext[1], acc_sp.at[dst_v[1]], add=True)
  if rem >= 3:
    pltpu.sync_copy(src_hbm.at[pl.ds(ebase(base + 2), B)], src_v[0])
    pltpu.sync_copy(dst_hbm.at[pl.ds(ebase(base + 2), B)], dst_v[2])
    start_gathers(base + 2, 0, 2)
    wait_gathers(0)
    compute(0, base + 2)
    pltpu.sync_copy(wext[0], acc_sp.at[dst_v[2]], add=True)

  plsc.subcore_barrier()
  pltpu.sync_copy(acc_sp.at[pl.ds(row0, ROWS_PER_TILE)],
                  out_hbm.at[c, pl.ds(row0, ROWS_PER_TILE)])


def _sc_edge_pass(hs, hd, he, src, dst, att):
  E = src.shape[0]
  nblk = E // B
  mesh = plsc.VectorSubcoreMesh(core_axis_name="c", subcore_axis_name="s")
  body = functools.partial(_sc_edge_body, nblk=nblk)
  f = pl.kernel(
      body,
      out_type=jax.ShapeDtypeStruct((NC, NP, CW), jnp.float32),
      mesh=mesh,
      compiler_params=pltpu.CompilerParams(
          needs_layout_passes=False, use_tc_tiling_on_sc=False),
      scratch_types=[
          pltpu.VMEM((CB, LANES), jnp.float32),
          pltpu.VMEM((B,), jnp.int32),
          pltpu.VMEM((B,), jnp.int32),
          pltpu.VMEM((B,), jnp.int32),
          pltpu.VMEM((B,), jnp.int32),
          pltpu.VMEM((B,), jnp.int32),
          pltpu.VMEM((B,), jnp.int32),
          pltpu.VMEM((B, C), jnp.float32),
          pltpu.VMEM((B, C), jnp.float32),
          pltpu.VMEM((B, C), jnp.float32),
          pltpu.VMEM((B, C), jnp.float32),
          pltpu.VMEM((B, C), jnp.float32),
          pltpu.VMEM((B, C), jnp.float32),
          pltpu.VMEM((B, CW), jnp.float32),
          pltpu.VMEM((B, CW), jnp.float32),
          pltpu.VMEM_SHARED((NP, CW), jnp.float32),
          pltpu.SemaphoreType.DMA,
          pltpu.SemaphoreType.DMA,
          pltpu.SemaphoreType.DMA,
          pltpu.SemaphoreType.DMA,
          pltpu.SemaphoreType.DMA,
      ],
  )
  return f(hs, hd, he, src, dst, att.reshape(CB, LANES))


def _mm_body(x_ref, w_ref, b_ref, o_ref):
  o_ref[...] = jnp.dot(x_ref[...], w_ref[...],
                       preferred_element_type=jnp.float32) + b_ref[...]


def _tc_matmul(x, w, b=None, bm=400):
  M, K = x.shape
  Kn = w.shape[1]
  if b is None:
    b = jnp.zeros((1, Kn), jnp.float32)
  else:
    b = b.reshape(1, Kn)
  grid = M // bm
  return pl.pallas_call(
      _mm_body,
      grid=(grid,),
      in_specs=[
          pl.BlockSpec((bm, K), lambda i: (i, 0)),
          pl.BlockSpec((K, Kn), lambda i: (0, 0)),
          pl.BlockSpec((1, Kn), lambda i: (0, 0)),
      ],
      out_specs=pl.BlockSpec((bm, Kn), lambda i: (i, 0)),
      out_shape=jax.ShapeDtypeStruct((M, Kn), jnp.float32),
  )(x, w, b)


def _fin1_body(a_ref, b_ref, o_ref):
  num = a_ref[0, :, :C] + a_ref[1, :, :C]
  den = a_ref[0, :, C:C + 1] + a_ref[1, :, C:C + 1]
  o_ref[...] = jnp.maximum(num / (den + 1e-16) + b_ref[...], 0.0)


def _finalize1(acc, bias, bm=400):
  return pl.pallas_call(
      _fin1_body,
      grid=(N // bm,),
      in_specs=[
          pl.BlockSpec((NC, bm, CW), lambda i: (0, i, 0)),
          pl.BlockSpec((1, C), lambda i: (0, 0)),
      ],
      out_specs=pl.BlockSpec((bm, C), lambda i: (i, 0)),
      out_shape=jax.ShapeDtypeStruct((N, C), jnp.float32),
  )(acc, bias.reshape(1, C))


def _fin2_body(a_ref, c_ref, b1_ref, b2_ref, o_ref):
  num1 = a_ref[0, :, :C] + a_ref[1, :, :C]
  den1 = a_ref[0, :, C:C + 1] + a_ref[1, :, C:C + 1]
  num2 = c_ref[0, :, :C] + c_ref[1, :, :C]
  den2 = c_ref[0, :, C:C + 1] + c_ref[1, :, C:C + 1]
  o_ref[...] = jnp.maximum(
      num1 / (den1 + 1e-16) + b1_ref[...] +
      num2 / (den2 + 1e-16) + b2_ref[...], 0.0)


def _finalize2(acc1, acc2, bias1, bias2, bm=400):
  return pl.pallas_call(
      _fin2_body,
      grid=(N // bm,),
      in_specs=[
          pl.BlockSpec((NC, bm, CW), lambda i: (0, i, 0)),
          pl.BlockSpec((NC, bm, CW), lambda i: (0, i, 0)),
          pl.BlockSpec((1, C), lambda i: (0, 0)),
          pl.BlockSpec((1, C), lambda i: (0, 0)),
      ],
      out_specs=pl.BlockSpec((bm, C), lambda i: (i, 0)),
      out_shape=jax.ShapeDtypeStruct((N, C), jnp.float32),
  )(acc1, acc2, bias1.reshape(1, C), bias2.reshape(1, C))


def kernel(x_adresse, x_batiment, x_parcelle, ea_ab, ea_ba, ea_bp, ea_pb,
           Wl, Wr, We, att, bias, Wlin, blin, ei_ab, ei_ba, ei_bp, ei_pb):
  xa, xb, xp = x_adresse, x_batiment, x_parcelle
  src_ab, dst_ab = ei_ab[0].astype(jnp.int32), ei_ab[1].astype(jnp.int32)
  src_ba, dst_ba = ei_ba[0].astype(jnp.int32), ei_ba[1].astype(jnp.int32)
  src_bp, dst_bp = ei_bp[0].astype(jnp.int32), ei_bp[1].astype(jnp.int32)
  src_pb, dst_pb = ei_pb[0].astype(jnp.int32), ei_pb[1].astype(jnp.int32)

  for l in range(2):
    wa = jnp.concatenate([Wl[l, 0], Wr[l, 1]], axis=1)
    wb = jnp.concatenate([Wr[l, 0], Wl[l, 1], Wl[l, 2], Wr[l, 3]], axis=1)
    wp = jnp.concatenate([Wr[l, 2], Wl[l, 3]], axis=1)
    ha = _tc_matmul(xa, wa)
    hb = _tc_matmul(xb, wb)
    hp = _tc_matmul(xp, wp)
    hs0, hd1 = ha[:, :C], ha[:, C:]
    hd0, hs1, hs2, hd3 = hb[:, :C], hb[:, C:2 * C], hb[:, 2 * C:3 * C], hb[:, 3 * C:]
    hd2, hs3 = hp[:, :C], hp[:, C:]

    he0 = _tc_matmul(ea_ab, We[l, 0], bm=1000)
    he1 = _tc_matmul(ea_ba, We[l, 1], bm=1000)
    he2 = _tc_matmul(ea_bp, We[l, 2], bm=1000)
    he3 = _tc_matmul(ea_pb, We[l, 3], bm=1000)

    acc0 = _sc_edge_pass(hs0, hd0, he0, src_ab, dst_ab, att[l, 0])
    acc1 = _sc_edge_pass(hs1, hd1, he1, src_ba, dst_ba, att[l, 1])
    acc2 = _sc_edge_pass(hs2, hd2, he2, src_bp, dst_bp, att[l, 2])
    acc3 = _sc_edge_pass(hs3, hd3, he3, src_pb, dst_pb, att[l, 3])

    xa = _finalize1(acc1, bias[l, 1])
    xp = _finalize1(acc2, bias[l, 2])
    xb = _finalize2(acc0, acc3, bias[l, 0], bias[l, 3])

  ya = _tc_matmul(xa, Wlin[0], blin[0])
  yb = _tc_matmul(xb, Wlin[1], blin[1])
  yp = _tc_matmul(xp, Wlin[2], blin[2])
  return (ya, yb, yp)
